# baseline (speedup 1.0000x reference)
import functools

import jax
import jax.numpy as jnp
from jax import lax
from jax.experimental import pallas as pl
from jax.experimental.pallas import tpu as pltpu
from jax.experimental.pallas import tpu_sc as plsc

N_NODES = 10000
N_EDGES = 320000
D_FEAT = 128
D_EDGE = 16
H_DIM = 5
E_DIM = 4
HID = 40
L_EC = 3
ALPHA = 0.5
HP = 8
PK = 16
EP = N_EDGES // PK
NP_ = N_NODES // PK

NC, NS = 2, 16
NW = NC * NS
EPW = N_EDGES // NW


@functools.lru_cache(maxsize=1)
def _build_sc_kernels():
    mesh = plsc.VectorSubcoreMesh(
        core_axis_name="c", subcore_axis_name="s",
        num_cores=NC, num_subcores=NS)

    CH = EPW // 2

    @functools.partial(
        pl.kernel,
        out_type=(jax.ShapeDtypeStruct((N_EDGES, HP), jnp.float32),
                  jax.ShapeDtypeStruct((N_EDGES, HP), jnp.float32)),
        mesh=mesh,
        scratch_types=[pltpu.VMEM((2, CH), jnp.int32),
                       pltpu.VMEM((2, CH, HP), jnp.float32),
                       pltpu.SemaphoreType.DMA,
                       pltpu.SemaphoreType.DMA],
        compiler_params=pltpu.CompilerParams(use_tc_tiling_on_sc=False),
    )
    def sc_gather(h_hbm, ei_hbm, hd_hbm, hs_hbm, idx_v, rows_v, sem0, sem1):
        wid = lax.axis_index("s") * NC + lax.axis_index("c")
        base = wid * EPW
        h8, hd8, hs8 = h_hbm, hd_hbm, hs_hbm
        sems = (sem0, sem1)
        tasks = [(1, 0, hd8), (1, 1, hd8), (0, 0, hs8), (0, 1, hs8)]
        copies = []
        for t, (row, c, _) in enumerate(tasks):
            s = t % 2
            pltpu.sync_copy(ei_hbm.at[row, pl.ds(base + c * CH, CH)],
                            idx_v.at[s])
            copies.append(
                pltpu.async_copy(h8.at[idx_v.at[s]], rows_v.at[s],
                                 sems[s]))
            if t > 0:
                prow, pc, pout = tasks[t - 1]
                copies[t - 1].wait()
                pltpu.sync_copy(rows_v.at[(t - 1) % 2],
                                pout.at[pl.ds(base + pc * CH, CH)])
        copies[3].wait()
        pltpu.sync_copy(rows_v.at[1], hs8.at[pl.ds(base + CH, CH)])

    @functools.partial(
        pl.kernel,
        out_type=jax.ShapeDtypeStruct((NC, N_NODES, HP), jnp.float32),
        mesh=mesh,
        scratch_types=[pltpu.VMEM((EPW,), jnp.int32),
                       pltpu.VMEM((EPW, HP), jnp.float32),
                       pltpu.VMEM_SHARED((N_NODES, HP), jnp.float32)],
        compiler_params=pltpu.CompilerParams(use_tc_tiling_on_sc=False),
    )
    def sc_scatter(et_hbm, ei_hbm, zeros_hbm, agg_hbm, idx_v, rows_v,
                   shared):
        cid = lax.axis_index("c")
        sid = lax.axis_index("s")
        wid = sid * NC + cid
        base = wid * EPW

        @pl.when(sid == 0)
        def _():
            pltpu.sync_copy(zeros_hbm, shared)

        plsc.subcore_barrier()
        pltpu.sync_copy(ei_hbm.at[1, pl.ds(base, EPW)], idx_v)
        pltpu.sync_copy(et_hbm.at[pl.ds(base, EPW)], rows_v)
        pltpu.sync_copy(rows_v, shared.at[idx_v], add=True)
        plsc.subcore_barrier()
        rps = N_NODES // NS
        pltpu.sync_copy(shared.at[pl.ds(sid * rps, rps)],
                        agg_hbm.at[cid, pl.ds(sid * rps, rps)])

    return sc_gather, sc_scatter


def _sc_gather(h_p, ei):
    hd, hs = _build_sc_kernels()[0](h_p.reshape(N_NODES, HP), ei)
    return hd.reshape(EP, PK * HP), hs.reshape(EP, PK * HP)


def _sc_scatter(et_p, ei, zeros_n):
    agg2 = _build_sc_kernels()[1](et_p.reshape(N_EDGES, HP), ei, zeros_n)
    return agg2.reshape(NC, NP_, PK * HP)


def _mm(a, b):
    return jax.lax.dot_general(
        a, b, (((1,), (0,)), ((), ())),
        preferred_element_type=jnp.float32)


def _ne_body(x_ref, w0_ref, w1_ref, o_ref):
    h = jnp.maximum(x_ref[...] @ w0_ref[...], 0.0)
    o_ref[...] = jnp.maximum(h @ w1_ref[...], 0.0)


def _ee_body(a_ref, w0_ref, w1_ref, o_ref):
    h = jnp.maximum(_mm(a_ref[...], w0_ref[...]), 0.0)
    o_ref[...] = jnp.maximum(_mm(h, w1_ref[...]), 0.0)


def _rel_body(hd_ref, hs_ref, ea_ref, w0_ref, b0_ref, w1_ref, b1_ref,
              w2_ref, b2_ref, et_ref, ean_ref):
    z = (_mm(hd_ref[...], w0_ref[0:128]) + _mm(hs_ref[...], w0_ref[128:256])
         + _mm(ea_ref[...], w0_ref[256:384]) + b0_ref[...])
    z = jnp.maximum(z, 0.0)
    z = jnp.maximum(_mm(z, w1_ref[...]) + b1_ref[...], 0.0)
    et = _mm(z, w2_ref[...]) + b2_ref[...]
    et_ref[...] = et
    ean_ref[...] = ALPHA * ea_ref[...] + (1.0 - ALPHA) * et


def _obj_body(h_ref, a0_ref, a1_ref, w0_ref, b0_ref, w1_ref, b1_ref,
              w2_ref, b2_ref, ho_ref):
    agg = a0_ref[...] + a1_ref[...]
    z = jnp.maximum(_mm(h_ref[...], w0_ref[0:128]) + _mm(agg, w0_ref[128:256])
                    + b0_ref[...], 0.0)
    z = jnp.maximum(_mm(z, w1_ref[...]) + b1_ref[...], 0.0)
    hn = _mm(z, w2_ref[...]) + b2_ref[...]
    ho_ref[...] = ALPHA * h_ref[...] + (1.0 - ALPHA) * hn


def _fin_body(e0_ref, e1_ref, e2_ref, e3_ref, w0_ref, b0_ref, w1_ref, b1_ref,
              w2_ref, b2_ref, o_ref):
    z = (_mm(e0_ref[...], w0_ref[0:128]) + _mm(e1_ref[...], w0_ref[128:256])
         + _mm(e2_ref[...], w0_ref[256:384])
         + _mm(e3_ref[...], w0_ref[384:512]) + b0_ref[...])
    z = jnp.maximum(z, 0.0)
    z = jnp.maximum(_mm(z, w1_ref[...]) + b1_ref[...], 0.0)
    o_ref[...] = jax.nn.sigmoid(_mm(z, w2_ref[...]) + b2_ref[...])


def _full(shape):
    return pl.BlockSpec(shape, lambda i: (0,) * len(shape))


def _rows(bs, w):
    return pl.BlockSpec((bs, w), lambda i: (i, 0))


BN = 2000
BPN = 625
BPE = 4000
BP8 = 4000


def _bd(w, k):
    return jnp.kron(jnp.eye(k, dtype=w.dtype), w)


def _padc(w, cols):
    return jnp.zeros((w.shape[0], cols), w.dtype).at[:, :w.shape[1]].set(w)


def _padr(w, rows):
    return jnp.zeros((rows, w.shape[1]), w.dtype).at[:w.shape[0], :].set(w)


def kernel(x, edge_index, edge_attr, ne_w0, ne_w1, ee_w0, ee_w1,
           rel_w0, rel_b0, rel_w1, rel_b1, rel_w2, rel_b2,
           obj_w0, obj_b0, obj_w1, obj_b1, obj_w2, obj_b2,
           w_w0, w_b0, w_w1, w_b1, w_w2, w_b2):
    f32 = jnp.float32
    ei = edge_index.astype(jnp.int32)

    ne_w1p = _padc(ne_w1, HP)
    ee_w0b = _bd(ee_w0, HP)
    ee_w1b = _bd(_padc(ee_w1, HP), HP)

    def tile(b):
        return jnp.tile(b, PK)[None]

    rel_w0b, rel_w1b, rel_w2b = [], [], []
    rel_b0t, rel_b1t, rel_b2t = [], [], []
    for l in range(L_EC):
        wd = _padr(rel_w0[l][0:H_DIM], HP)
        ws = _padr(rel_w0[l][H_DIM:2 * H_DIM], HP)
        we = _padr(rel_w0[l][2 * H_DIM:], HP)
        rel_w0b.append(jnp.concatenate(
            [_bd(wd, PK), _bd(ws, PK), _bd(we, PK)], axis=0))
        rel_w1b.append(_bd(rel_w1[l], PK))
        rel_w2b.append(_bd(_padc(rel_w2[l], HP), PK))
        rel_b0t.append(tile(rel_b0[l]))
        rel_b1t.append(tile(rel_b1[l]))
        rel_b2t.append(tile(jnp.concatenate(
            [rel_b2[l], jnp.zeros((HP - E_DIM,), f32)])))

    obj_w0b, obj_w1b, obj_w2b = [], [], []
    obj_b0t, obj_b1t, obj_b2t = [], [], []
    for l in range(L_EC):
        wh = _padr(obj_w0[l][0:H_DIM], HP)
        wa = _padr(obj_w0[l][H_DIM:], HP)
        obj_w0b.append(jnp.concatenate(
            [_bd(wh, PK), _bd(wa, PK)], axis=0))
        obj_w1b.append(_bd(obj_w1[l], PK))
        obj_w2b.append(_bd(_padc(obj_w2[l], HP), PK))
        obj_b0t.append(tile(obj_b0[l]))
        obj_b1t.append(tile(obj_b1[l]))
        obj_b2t.append(tile(jnp.concatenate(
            [obj_b2[l], jnp.zeros((HP - H_DIM,), f32)])))

    w_w0b = jnp.concatenate(
        [_bd(_padr(w_w0[k * E_DIM:(k + 1) * E_DIM], HP), PK)
         for k in range(L_EC + 1)], axis=0)
    w_w1b = _bd(w_w1, PK)
    w_w2b = _bd(w_w2, PK)
    w_b0t = tile(w_b0)
    w_b1t = tile(w_b1)
    w_b2t = tile(w_b2)
    zeros_n = jnp.zeros((N_NODES, HP), f32)

    h_p = pl.pallas_call(
        _ne_body,
        grid=(N_NODES // BN,),
        in_specs=[_rows(BN, D_FEAT), _full((D_FEAT, HID)), _full((HID, HP))],
        out_specs=_rows(BN, HP),
        out_shape=jax.ShapeDtypeStruct((N_NODES, HP), f32),
    )(x, ne_w0, ne_w1p).reshape(NP_, PK * HP)

    ea_p = pl.pallas_call(
        _ee_body,
        grid=(N_EDGES // HP // BP8,),
        in_specs=[_rows(BP8, HP * D_EDGE), _full((HP * D_EDGE, HP * HID)),
                  _full((HP * HID, HP * HP))],
        out_specs=_rows(BP8, HP * HP),
        out_shape=jax.ShapeDtypeStruct((N_EDGES // HP, HP * HP), f32),
    )(edge_attr.reshape(N_EDGES // HP, HP * D_EDGE), ee_w0b,
      ee_w1b).reshape(EP, PK * HP)

    eas = [ea_p]
    for l in range(L_EC):
        hd, hs = _sc_gather(h_p, ei)
        et_p, ea_p = pl.pallas_call(
            _rel_body,
            grid=(EP // BPE,),
            in_specs=[_rows(BPE, PK * HP)] * 3 +
                     [_full((3 * PK * HP, PK * HID)), _full((1, PK * HID)),
                      _full((PK * HID, PK * HID)), _full((1, PK * HID)),
                      _full((PK * HID, PK * HP)), _full((1, PK * HP))],
            out_specs=(_rows(BPE, PK * HP), _rows(BPE, PK * HP)),
            out_shape=(jax.ShapeDtypeStruct((EP, PK * HP), f32),
                       jax.ShapeDtypeStruct((EP, PK * HP), f32)),
        )(hd, hs, eas[-1], rel_w0b[l], rel_b0t[l], rel_w1b[l], rel_b1t[l],
          rel_w2b[l], rel_b2t[l])

        agg2 = _sc_scatter(et_p, ei, zeros_n)

        h_p = pl.pallas_call(
            _obj_body,
            grid=(NP_ // BPN,),
            in_specs=[_rows(BPN, PK * HP), _rows(BPN, PK * HP),
                      _rows(BPN, PK * HP),
                      _full((2 * PK * HP, PK * HID)), _full((1, PK * HID)),
                      _full((PK * HID, PK * HID)), _full((1, PK * HID)),
                      _full((PK * HID, PK * HP)), _full((1, PK * HP))],
            out_specs=_rows(BPN, PK * HP),
            out_shape=jax.ShapeDtypeStruct((NP_, PK * HP), f32),
        )(h_p, agg2[0], agg2[1], obj_w0b[l], obj_b0t[l], obj_w1b[l],
          obj_b1t[l], obj_w2b[l], obj_b2t[l])
        eas.append(ea_p)

    out = pl.pallas_call(
        _fin_body,
        grid=(EP // BPE,),
        in_specs=[_rows(BPE, PK * HP)] * 4 +
                 [_full((4 * PK * HP, PK * HID)), _full((1, PK * HID)),
                  _full((PK * HID, PK * HID)), _full((1, PK * HID)),
                  _full((PK * HID, PK)), _full((1, PK))],
        out_specs=_rows(BPE, PK),
        out_shape=jax.ShapeDtypeStruct((EP, PK), f32),
    )(eas[0], eas[1], eas[2], eas[3], w_w0b, w_b0t, w_w1b, w_b1t,
      w_w2b, w_b2t)
    return out.reshape(N_EDGES, 1)


# speedup vs baseline: 1.0107x; 1.0107x over previous
import functools

import jax
import jax.numpy as jnp
from jax import lax
from jax.experimental import pallas as pl
from jax.experimental.pallas import tpu as pltpu
from jax.experimental.pallas import tpu_sc as plsc

N_NODES = 10000
N_EDGES = 320000
D_FEAT = 128
D_EDGE = 16
H_DIM = 5
E_DIM = 4
HID = 40
L_EC = 3
ALPHA = 0.5
HP = 8
PK = 16
EP = N_EDGES // PK
NP_ = N_NODES // PK

NC, NS = 2, 16
NW = NC * NS
EPW = N_EDGES // NW


@functools.lru_cache(maxsize=1)
def _build_sc_kernels():
    mesh = plsc.VectorSubcoreMesh(
        core_axis_name="c", subcore_axis_name="s",
        num_cores=NC, num_subcores=NS)

    CH = EPW // 2

    @functools.partial(
        pl.kernel,
        out_type=(jax.ShapeDtypeStruct((N_EDGES, HP), jnp.float32),
                  jax.ShapeDtypeStruct((N_EDGES, HP), jnp.float32)),
        mesh=mesh,
        scratch_types=[pltpu.VMEM((2, CH), jnp.int32),
                       pltpu.VMEM((2, CH, HP), jnp.float32),
                       pltpu.SemaphoreType.DMA,
                       pltpu.SemaphoreType.DMA],
        compiler_params=pltpu.CompilerParams(use_tc_tiling_on_sc=False),
    )
    def sc_gather(h_hbm, ei_hbm, hd_hbm, hs_hbm, idx_v, rows_v, sem0, sem1):
        wid = lax.axis_index("s") * NC + lax.axis_index("c")
        base = wid * EPW
        h8, hd8, hs8 = h_hbm, hd_hbm, hs_hbm
        sems = (sem0, sem1)
        tasks = [(1, 0, hd8), (1, 1, hd8), (0, 0, hs8), (0, 1, hs8)]
        copies = []
        for t, (row, c, _) in enumerate(tasks):
            s = t % 2
            pltpu.sync_copy(ei_hbm.at[row, pl.ds(base + c * CH, CH)],
                            idx_v.at[s])
            copies.append(
                pltpu.async_copy(h8.at[idx_v.at[s]], rows_v.at[s],
                                 sems[s]))
            if t > 0:
                prow, pc, pout = tasks[t - 1]
                copies[t - 1].wait()
                pltpu.sync_copy(rows_v.at[(t - 1) % 2],
                                pout.at[pl.ds(base + pc * CH, CH)])
        copies[3].wait()
        pltpu.sync_copy(rows_v.at[1], hs8.at[pl.ds(base + CH, CH)])

    @functools.partial(
        pl.kernel,
        out_type=jax.ShapeDtypeStruct((NC, N_NODES, HP), jnp.float32),
        mesh=mesh,
        scratch_types=[pltpu.VMEM((EPW,), jnp.int32),
                       pltpu.VMEM((EPW, HP), jnp.float32),
                       pltpu.VMEM_SHARED((N_NODES, HP), jnp.float32)],
        compiler_params=pltpu.CompilerParams(use_tc_tiling_on_sc=False),
    )
    def sc_scatter(et_hbm, ei_hbm, zeros_hbm, agg_hbm, idx_v, rows_v,
                   shared):
        cid = lax.axis_index("c")
        sid = lax.axis_index("s")
        wid = sid * NC + cid
        base = wid * EPW

        @pl.when(sid == 0)
        def _():
            pltpu.sync_copy(zeros_hbm, shared)

        plsc.subcore_barrier()
        pltpu.sync_copy(ei_hbm.at[1, pl.ds(base, EPW)], idx_v)
        pltpu.sync_copy(et_hbm.at[pl.ds(base, EPW)], rows_v)
        pltpu.sync_copy(rows_v, shared.at[idx_v], add=True)
        plsc.subcore_barrier()
        rps = N_NODES // NS
        pltpu.sync_copy(shared.at[pl.ds(sid * rps, rps)],
                        agg_hbm.at[cid, pl.ds(sid * rps, rps)])

    return sc_gather, sc_scatter


def _sc_gather(h_p, ei):
    hd, hs = _build_sc_kernels()[0](h_p.reshape(N_NODES, HP), ei)
    return hd.reshape(EP, PK * HP), hs.reshape(EP, PK * HP)


def _sc_scatter(et_p, ei, zeros_n):
    agg2 = _build_sc_kernels()[1](et_p.reshape(N_EDGES, HP), ei, zeros_n)
    return agg2.reshape(NC, NP_, PK * HP)


def _mm(a, b):
    return jax.lax.dot_general(
        a, b, (((1,), (0,)), ((), ())),
        preferred_element_type=jnp.float32)


def _ne_body(x_ref, w0_ref, w1_ref, o_ref):
    h = jnp.maximum(x_ref[...] @ w0_ref[...], 0.0)
    o_ref[...] = jnp.maximum(h @ w1_ref[...], 0.0)


def _ee_body(a_ref, w0_ref, w1_ref, o_ref):
    h = jnp.maximum(_mm(a_ref[...], w0_ref[...]), 0.0)
    o_ref[...] = jnp.maximum(_mm(h, w1_ref[...]), 0.0)


def _rel_body(hd_ref, hs_ref, ea_ref, w0_ref, b0_ref, w1_ref, b1_ref,
              w2_ref, b2_ref, et_ref, ean_ref):
    z = (_mm(hd_ref[...], w0_ref[0:128]) + _mm(hs_ref[...], w0_ref[128:256])
         + _mm(ea_ref[...], w0_ref[256:384]) + b0_ref[...])
    z = jnp.maximum(z, 0.0)
    z = jnp.maximum(_mm(z, w1_ref[...]) + b1_ref[...], 0.0)
    et = _mm(z, w2_ref[...]) + b2_ref[...]
    et_ref[...] = et
    ean_ref[...] = ALPHA * ea_ref[...] + (1.0 - ALPHA) * et


def _obj_body(h_ref, a0_ref, a1_ref, w0_ref, b0_ref, w1_ref, b1_ref,
              w2_ref, b2_ref, ho_ref):
    agg = a0_ref[...] + a1_ref[...]
    z = jnp.maximum(_mm(h_ref[...], w0_ref[0:128]) + _mm(agg, w0_ref[128:256])
                    + b0_ref[...], 0.0)
    z = jnp.maximum(_mm(z, w1_ref[...]) + b1_ref[...], 0.0)
    hn = _mm(z, w2_ref[...]) + b2_ref[...]
    ho_ref[...] = ALPHA * h_ref[...] + (1.0 - ALPHA) * hn


def _fin_body(e0_ref, e1_ref, e2_ref, e3_ref, w0_ref, b0_ref, w1_ref, b1_ref,
              w2_ref, b2_ref, o_ref):
    z = (_mm(e0_ref[...], w0_ref[0:128]) + _mm(e1_ref[...], w0_ref[128:256])
         + _mm(e2_ref[...], w0_ref[256:384])
         + _mm(e3_ref[...], w0_ref[384:512]) + b0_ref[...])
    z = jnp.maximum(z, 0.0)
    z = jnp.maximum(_mm(z, w1_ref[...]) + b1_ref[...], 0.0)
    o_ref[...] = jax.nn.sigmoid(_mm(z, w2_ref[...]) + b2_ref[...])


def _full(shape):
    return pl.BlockSpec(shape, lambda i: (0,) * len(shape))


def _rows(bs, w):
    return pl.BlockSpec((bs, w), lambda i: (i, 0))


BN = 2000
BPN = 625
BPE = 2000
BP8 = 4000


def _bd(w, k):
    return jnp.kron(jnp.eye(k, dtype=w.dtype), w)


def _padc(w, cols):
    return jnp.zeros((w.shape[0], cols), w.dtype).at[:, :w.shape[1]].set(w)


def _padr(w, rows):
    return jnp.zeros((rows, w.shape[1]), w.dtype).at[:w.shape[0], :].set(w)


def kernel(x, edge_index, edge_attr, ne_w0, ne_w1, ee_w0, ee_w1,
           rel_w0, rel_b0, rel_w1, rel_b1, rel_w2, rel_b2,
           obj_w0, obj_b0, obj_w1, obj_b1, obj_w2, obj_b2,
           w_w0, w_b0, w_w1, w_b1, w_w2, w_b2):
    f32 = jnp.float32
    ei = edge_index.astype(jnp.int32)

    ne_w1p = _padc(ne_w1, HP)
    ee_w0b = _bd(ee_w0, HP)
    ee_w1b = _bd(_padc(ee_w1, HP), HP)

    def tile(b):
        return jnp.tile(b, PK)[None]

    rel_w0b, rel_w1b, rel_w2b = [], [], []
    rel_b0t, rel_b1t, rel_b2t = [], [], []
    for l in range(L_EC):
        wd = _padr(rel_w0[l][0:H_DIM], HP)
        ws = _padr(rel_w0[l][H_DIM:2 * H_DIM], HP)
        we = _padr(rel_w0[l][2 * H_DIM:], HP)
        rel_w0b.append(jnp.concatenate(
            [_bd(wd, PK), _bd(ws, PK), _bd(we, PK)], axis=0))
        rel_w1b.append(_bd(rel_w1[l], PK))
        rel_w2b.append(_bd(_padc(rel_w2[l], HP), PK))
        rel_b0t.append(tile(rel_b0[l]))
        rel_b1t.append(tile(rel_b1[l]))
        rel_b2t.append(tile(jnp.concatenate(
            [rel_b2[l], jnp.zeros((HP - E_DIM,), f32)])))

    obj_w0b, obj_w1b, obj_w2b = [], [], []
    obj_b0t, obj_b1t, obj_b2t = [], [], []
    for l in range(L_EC):
        wh = _padr(obj_w0[l][0:H_DIM], HP)
        wa = _padr(obj_w0[l][H_DIM:], HP)
        obj_w0b.append(jnp.concatenate(
            [_bd(wh, PK), _bd(wa, PK)], axis=0))
        obj_w1b.append(_bd(obj_w1[l], PK))
        obj_w2b.append(_bd(_padc(obj_w2[l], HP), PK))
        obj_b0t.append(tile(obj_b0[l]))
        obj_b1t.append(tile(obj_b1[l]))
        obj_b2t.append(tile(jnp.concatenate(
            [obj_b2[l], jnp.zeros((HP - H_DIM,), f32)])))

    w_w0b = jnp.concatenate(
        [_bd(_padr(w_w0[k * E_DIM:(k + 1) * E_DIM], HP), PK)
         for k in range(L_EC + 1)], axis=0)
    w_w1b = _bd(w_w1, PK)
    w_w2b = _bd(w_w2, PK)
    w_b0t = tile(w_b0)
    w_b1t = tile(w_b1)
    w_b2t = tile(w_b2)
    zeros_n = jnp.zeros((N_NODES, HP), f32)

    h_p = pl.pallas_call(
        _ne_body,
        grid=(N_NODES // BN,),
        in_specs=[_rows(BN, D_FEAT), _full((D_FEAT, HID)), _full((HID, HP))],
        out_specs=_rows(BN, HP),
        out_shape=jax.ShapeDtypeStruct((N_NODES, HP), f32),
    )(x, ne_w0, ne_w1p).reshape(NP_, PK * HP)

    ea_p = pl.pallas_call(
        _ee_body,
        grid=(N_EDGES // HP // BP8,),
        in_specs=[_rows(BP8, HP * D_EDGE), _full((HP * D_EDGE, HP * HID)),
                  _full((HP * HID, HP * HP))],
        out_specs=_rows(BP8, HP * HP),
        out_shape=jax.ShapeDtypeStruct((N_EDGES // HP, HP * HP), f32),
    )(edge_attr.reshape(N_EDGES // HP, HP * D_EDGE), ee_w0b,
      ee_w1b).reshape(EP, PK * HP)

    eas = [ea_p]
    for l in range(L_EC):
        hd, hs = _sc_gather(h_p, ei)
        et_p, ea_p = pl.pallas_call(
            _rel_body,
            grid=(EP // BPE,),
            in_specs=[_rows(BPE, PK * HP)] * 3 +
                     [_full((3 * PK * HP, PK * HID)), _full((1, PK * HID)),
                      _full((PK * HID, PK * HID)), _full((1, PK * HID)),
                      _full((PK * HID, PK * HP)), _full((1, PK * HP))],
            out_specs=(_rows(BPE, PK * HP), _rows(BPE, PK * HP)),
            out_shape=(jax.ShapeDtypeStruct((EP, PK * HP), f32),
                       jax.ShapeDtypeStruct((EP, PK * HP), f32)),
        )(hd, hs, eas[-1], rel_w0b[l], rel_b0t[l], rel_w1b[l], rel_b1t[l],
          rel_w2b[l], rel_b2t[l])

        agg2 = _sc_scatter(et_p, ei, zeros_n)

        h_p = pl.pallas_call(
            _obj_body,
            grid=(NP_ // BPN,),
            in_specs=[_rows(BPN, PK * HP), _rows(BPN, PK * HP),
                      _rows(BPN, PK * HP),
                      _full((2 * PK * HP, PK * HID)), _full((1, PK * HID)),
                      _full((PK * HID, PK * HID)), _full((1, PK * HID)),
                      _full((PK * HID, PK * HP)), _full((1, PK * HP))],
            out_specs=_rows(BPN, PK * HP),
            out_shape=jax.ShapeDtypeStruct((NP_, PK * HP), f32),
        )(h_p, agg2[0], agg2[1], obj_w0b[l], obj_b0t[l], obj_w1b[l],
          obj_b1t[l], obj_w2b[l], obj_b2t[l])
        eas.append(ea_p)

    out = pl.pallas_call(
        _fin_body,
        grid=(EP // BPE,),
        in_specs=[_rows(BPE, PK * HP)] * 4 +
                 [_full((4 * PK * HP, PK * HID)), _full((1, PK * HID)),
                  _full((PK * HID, PK * HID)), _full((1, PK * HID)),
                  _full((PK * HID, PK)), _full((1, PK))],
        out_specs=_rows(BPE, PK),
        out_shape=jax.ShapeDtypeStruct((EP, PK), f32),
    )(eas[0], eas[1], eas[2], eas[3], w_w0b, w_b0t, w_w1b, w_b1t,
      w_w2b, w_b2t)
    return out.reshape(N_EDGES, 1)
